# Initial kernel scaffold; baseline (speedup 1.0000x reference)
import jax
import jax.numpy as jnp
from jax.experimental import pallas as pl


def kernel(ids, vals, weight, bias):
    raise NotImplementedError("write your pallas kernel here")


# speedup vs baseline: 1.4315x; 1.4315x over previous
import functools

import jax
import jax.numpy as jnp
from jax import lax
from jax.experimental import pallas as pl
from jax.experimental.pallas import tpu as pltpu
from jax.experimental.pallas import tpu_sc as plsc

NC = 2
NS = 16
NW = NC * NS
L = 16


def _make_sc_kernel(B, F):
    assert (B * F) % (NW * 128) == 0
    rows_pt = B // NW
    flat_pt = rows_pt * F
    nrow = flat_pt // 128
    groups = rows_pt // L

    mesh = plsc.VectorSubcoreMesh(core_axis_name="c", subcore_axis_name="s")

    @functools.partial(
        pl.kernel,
        out_type=jax.ShapeDtypeStruct((B,), jnp.float32),
        mesh=mesh,
        scratch_types=[
            pltpu.VMEM((nrow, 128), jnp.int32),
            pltpu.VMEM((flat_pt,), jnp.float32),
            pltpu.VMEM((flat_pt,), jnp.float32),
            pltpu.VMEM((L,), jnp.float32),
            pltpu.VMEM((rows_pt,), jnp.float32),
            pltpu.SemaphoreType.DMA,
        ],
    )
    def sc_kernel(w_hbm, ids_hbm, vals_hbm, bias_hbm, out_hbm,
                  idx_v, vals_v, wg_v, bias_v, out_v, sem):
        wid = lax.axis_index("s") * NC + lax.axis_index("c")
        r0 = wid * nrow

        pltpu.sync_copy(ids_hbm.at[pl.ds(r0, nrow)], idx_v)
        pltpu.sync_copy(vals_hbm.at[pl.ds(wid * flat_pt, flat_pt)], vals_v)
        pltpu.sync_copy(bias_hbm, bias_v)

        def fire(j, carry):
            pltpu.async_copy(w_hbm.at[idx_v.at[j]], wg_v.at[pl.ds(j * 128, 128)], sem)
            return carry

        lax.fori_loop(0, nrow, fire, 0)

        def drain(j, carry):
            pltpu.make_async_copy(
                w_hbm.at[idx_v.at[j]], wg_v.at[pl.ds(j * 128, 128)], sem).wait()
            return carry

        lax.fori_loop(0, nrow, drain, 0)

        bias_vec = bias_v[...]

        def group(g, carry):
            b0 = g * L
            acc = bias_vec
            for f in range(F):
                o = f * rows_pt + b0
                acc = acc + wg_v[pl.ds(o, L)] * vals_v[pl.ds(o, L)]
            out_v[pl.ds(b0, L)] = acc
            return carry

        lax.fori_loop(0, groups, group, 0)

        pltpu.sync_copy(out_v, out_hbm.at[pl.ds(wid * rows_pt, rows_pt)])

    return sc_kernel


def kernel(ids, vals, weight, bias):
    B, F = ids.shape
    rows_pt = B // NW
    w_flat = weight.reshape(-1)
    ids2 = ids.reshape(NW, rows_pt, F).transpose(0, 2, 1).reshape(-1, 128)
    vals2 = vals.reshape(NW, rows_pt, F).transpose(0, 2, 1).reshape(-1)
    bias16 = jnp.broadcast_to(bias.astype(jnp.float32), (L,))
    sc = _make_sc_kernel(B, F)
    return sc(w_flat, ids2, vals2, bias16)


# speedup vs baseline: 1.5949x; 1.1141x over previous
import functools

import jax
import jax.numpy as jnp
from jax import lax
from jax.experimental import pallas as pl
from jax.experimental.pallas import tpu as pltpu
from jax.experimental.pallas import tpu_sc as plsc

NC = 2
NS = 16
NW = NC * NS
L = 16


def _make_sc_kernel(B, F):
    rows_pt = B // NW
    assert rows_pt % 128 == 0
    cpf = rows_pt // 128
    nchunk = F * cpf
    groups = rows_pt // L

    mesh = plsc.VectorSubcoreMesh(core_axis_name="c", subcore_axis_name="s")

    @functools.partial(
        pl.kernel,
        out_type=jax.ShapeDtypeStruct((B,), jnp.float32),
        mesh=mesh,
        scratch_types=[
            pltpu.VMEM((F, rows_pt), jnp.int32),
            pltpu.VMEM((F, rows_pt), jnp.float32),
            pltpu.VMEM((F, rows_pt), jnp.float32),
            pltpu.VMEM((L,), jnp.float32),
            pltpu.VMEM((rows_pt,), jnp.float32),
            pltpu.SemaphoreType.DMA,
        ],
    )
    def sc_kernel(w_hbm, ids_hbm, vals_hbm, bias_hbm, out_hbm,
                  idx_v, vals_v, wg_v, bias_v, out_v, sem):
        wid = lax.axis_index("s") * NC + lax.axis_index("c")
        b0 = wid * rows_pt

        pltpu.sync_copy(ids_hbm.at[:, pl.ds(b0, rows_pt)], idx_v)
        pltpu.sync_copy(vals_hbm.at[:, pl.ds(b0, rows_pt)], vals_v)
        pltpu.sync_copy(bias_hbm, bias_v)

        def fire(j, carry):
            f = j // cpf
            c = (j % cpf) * 128
            pltpu.async_copy(
                w_hbm.at[idx_v.at[f, pl.ds(c, 128)]],
                wg_v.at[f, pl.ds(c, 128)], sem)
            return carry

        lax.fori_loop(0, nchunk, fire, 0)

        def drain(j, carry):
            f = j // cpf
            c = (j % cpf) * 128
            pltpu.make_async_copy(
                w_hbm.at[idx_v.at[f, pl.ds(c, 128)]],
                wg_v.at[f, pl.ds(c, 128)], sem).wait()
            return carry

        lax.fori_loop(0, nchunk, drain, 0)

        bias_vec = bias_v[...]

        def group(g, carry):
            o = g * L
            acc = bias_vec
            for f in range(F):
                acc = acc + wg_v[f, pl.ds(o, L)] * vals_v[f, pl.ds(o, L)]
            out_v[pl.ds(o, L)] = acc
            return carry

        lax.fori_loop(0, groups, group, 0)

        pltpu.sync_copy(out_v, out_hbm.at[pl.ds(b0, rows_pt)])

    return sc_kernel


def kernel(ids, vals, weight, bias):
    B, F = ids.shape
    w_flat = weight.reshape(-1)
    ids_t = ids.T
    vals_t = vals.T
    bias16 = jnp.broadcast_to(bias.astype(jnp.float32), (L,))
    sc = _make_sc_kernel(B, F)
    return sc(w_flat, ids_t, vals_t, bias16)


# speedup vs baseline: 1.6916x; 1.0606x over previous
import functools

import jax
import jax.numpy as jnp
from jax import lax
from jax.experimental import pallas as pl
from jax.experimental.pallas import tpu as pltpu
from jax.experimental.pallas import tpu_sc as plsc

NC = 2
NS = 16
NW = NC * NS
L = 16


def _make_sc_kernel(B, F):
    rows_pt = B // NW
    assert rows_pt % 128 == 0
    cpf = rows_pt // 128
    nchunk = F * cpf
    groups = rows_pt // L

    mesh = plsc.VectorSubcoreMesh(core_axis_name="c", subcore_axis_name="s")

    V = 1000000
    slab = (V // NS) & ~7
    tail0 = V - NS * slab

    @functools.partial(
        pl.kernel,
        out_type=jax.ShapeDtypeStruct((B,), jnp.float32),
        mesh=mesh,
        scratch_types=[
            pltpu.VMEM_SHARED((V,), jnp.float32),
            pltpu.VMEM((slab // 4,), jnp.float32),
            pltpu.VMEM((F, rows_pt), jnp.int32),
            pltpu.VMEM((F, rows_pt), jnp.float32),
            pltpu.VMEM((F, rows_pt), jnp.float32),
            pltpu.VMEM((L,), jnp.float32),
            pltpu.VMEM((rows_pt,), jnp.float32),
            pltpu.SemaphoreType.DMA,
        ],
    )
    def sc_kernel(w_hbm, ids_hbm, vals_hbm, bias_hbm, out_hbm,
                  spw, bounce, idx_v, vals_v, wg_v, bias_v, out_v, sem):
        cid = lax.axis_index("c")
        sid = lax.axis_index("s")
        wid = sid * NC + cid
        b0 = wid * rows_pt

        s0 = sid * slab
        q = slab // 4
        pltpu.async_copy(w_hbm.at[pl.ds(s0, q)], bounce, sem)
        pltpu.sync_copy(ids_hbm.at[:, pl.ds(b0, rows_pt)], idx_v)
        pltpu.sync_copy(vals_hbm.at[:, pl.ds(b0, rows_pt)], vals_v)
        pltpu.sync_copy(bias_hbm, bias_v)
        pltpu.make_async_copy(w_hbm.at[pl.ds(s0, q)], bounce, sem).wait()
        pltpu.sync_copy(bounce, spw.at[pl.ds(s0, q)])
        for r in range(1, 4):
            pltpu.async_copy(w_hbm.at[pl.ds(s0 + r * q, q)], bounce, sem).wait()
            pltpu.sync_copy(bounce, spw.at[pl.ds(s0 + r * q, q)])

        @pl.when(sid == 0)
        def _():
            pltpu.async_copy(w_hbm.at[pl.ds(NS * slab, tail0)],
                             bounce.at[pl.ds(0, tail0)], sem).wait()
            pltpu.sync_copy(bounce.at[pl.ds(0, tail0)],
                            spw.at[pl.ds(NS * slab, tail0)])

        plsc.subcore_barrier()

        def fire(j, carry):
            f = j // cpf
            c = (j % cpf) * 128
            pltpu.async_copy(
                spw.at[idx_v.at[f, pl.ds(c, 128)]],
                wg_v.at[f, pl.ds(c, 128)], sem)
            return carry

        lax.fori_loop(0, nchunk, fire, 0)

        def drain(j, carry):
            f = j // cpf
            c = (j % cpf) * 128
            pltpu.make_async_copy(
                spw.at[idx_v.at[f, pl.ds(c, 128)]],
                wg_v.at[f, pl.ds(c, 128)], sem).wait()
            return carry

        lax.fori_loop(0, nchunk, drain, 0)

        bias_vec = bias_v[...]

        def group(g, carry):
            o = g * L
            acc = bias_vec
            for f in range(F):
                acc = acc + wg_v[f, pl.ds(o, L)] * vals_v[f, pl.ds(o, L)]
            out_v[pl.ds(o, L)] = acc
            return carry

        lax.fori_loop(0, groups, group, 0)

        pltpu.sync_copy(out_v, out_hbm.at[pl.ds(b0, rows_pt)])

    return sc_kernel


def kernel(ids, vals, weight, bias):
    B, F = ids.shape
    w_flat = weight.reshape(-1)
    ids_t = ids.T
    vals_t = vals.T
    bias16 = jnp.broadcast_to(bias.astype(jnp.float32), (L,))
    sc = _make_sc_kernel(B, F)
    return sc(w_flat, ids_t, vals_t, bias16)


# speedup vs baseline: 1.7407x; 1.0290x over previous
import functools

import jax
import jax.numpy as jnp
from jax import lax
from jax.experimental import pallas as pl
from jax.experimental.pallas import tpu as pltpu
from jax.experimental.pallas import tpu_sc as plsc

NC = 2
NS = 16
NW = NC * NS
L = 16


def _make_sc_kernel(B, F):
    rows_pt = B // NW
    assert rows_pt % 128 == 0
    cpf = rows_pt // 128
    nchunk = F * cpf
    groups = rows_pt // L

    mesh = plsc.VectorSubcoreMesh(core_axis_name="c", subcore_axis_name="s")

    V = 1000000
    slab = (V // NS) & ~63
    tail0 = V - NS * slab

    @functools.partial(
        pl.kernel,
        out_type=jax.ShapeDtypeStruct((B,), jnp.float32),
        mesh=mesh,
        scratch_types=[
            pltpu.VMEM_SHARED((V,), jnp.float32),
            pltpu.VMEM((slab // 8,), jnp.float32),
            pltpu.VMEM((slab // 8,), jnp.float32),
            pltpu.VMEM((F, rows_pt), jnp.int32),
            pltpu.VMEM((F, rows_pt), jnp.float32),
            pltpu.VMEM((F, rows_pt), jnp.float32),
            pltpu.VMEM((L,), jnp.float32),
            pltpu.VMEM((rows_pt,), jnp.float32),
            pltpu.SemaphoreType.DMA,
        ],
    )
    def sc_kernel(w_hbm, ids_hbm, vals_hbm, bias_hbm, out_hbm,
                  spw, bounce, bounce2, idx_v, vals_v, wg_v, bias_v, out_v, sem):
        cid = lax.axis_index("c")
        sid = lax.axis_index("s")
        wid = sid * NC + cid
        b0 = wid * rows_pt

        s0 = sid * slab
        q = slab // 8
        bufs = (bounce, bounce2)
        sem2 = sem
        pltpu.async_copy(w_hbm.at[pl.ds(s0, q)], bounce, sem)
        pltpu.async_copy(w_hbm.at[pl.ds(s0 + q, q)], bounce2, sem2)
        pltpu.sync_copy(ids_hbm.at[:, pl.ds(b0, rows_pt)], idx_v)
        pltpu.sync_copy(vals_hbm.at[:, pl.ds(b0, rows_pt)], vals_v)
        pltpu.sync_copy(bias_hbm, bias_v)
        for r in range(8):
            buf = bufs[r % 2]
            pltpu.make_async_copy(w_hbm.at[pl.ds(s0 + r * q, q)], buf, sem).wait()
            if r + 2 < 8:
                pltpu.async_copy(w_hbm.at[pl.ds(s0 + (r + 2) * q, q)], buf, sem)
            pltpu.sync_copy(buf, spw.at[pl.ds(s0 + r * q, q)])

        @pl.when(sid == 0)
        def _():
            pltpu.async_copy(w_hbm.at[pl.ds(NS * slab, tail0)],
                             bounce.at[pl.ds(0, tail0)], sem).wait()
            pltpu.sync_copy(bounce.at[pl.ds(0, tail0)],
                            spw.at[pl.ds(NS * slab, tail0)])

        plsc.subcore_barrier()

        def fire(j, carry):
            f = j // cpf
            c = (j % cpf) * 128
            pltpu.async_copy(
                spw.at[idx_v.at[f, pl.ds(c, 128)]],
                wg_v.at[f, pl.ds(c, 128)], sem)
            return carry

        lax.fori_loop(0, nchunk, fire, 0)

        def drain(j, carry):
            f = j // cpf
            c = (j % cpf) * 128
            pltpu.make_async_copy(
                spw.at[idx_v.at[f, pl.ds(c, 128)]],
                wg_v.at[f, pl.ds(c, 128)], sem).wait()
            return carry

        lax.fori_loop(0, nchunk, drain, 0)

        bias_vec = bias_v[...]

        def group(g, carry):
            o = g * L
            acc = bias_vec
            for f in range(F):
                acc = acc + wg_v[f, pl.ds(o, L)] * vals_v[f, pl.ds(o, L)]
            out_v[pl.ds(o, L)] = acc
            return carry

        lax.fori_loop(0, groups, group, 0)

        pltpu.sync_copy(out_v, out_hbm.at[pl.ds(b0, rows_pt)])

    return sc_kernel


def kernel(ids, vals, weight, bias):
    B, F = ids.shape
    w_flat = weight.reshape(-1)
    ids_t = ids.T
    vals_t = vals.T
    bias16 = jnp.broadcast_to(bias.astype(jnp.float32), (L,))
    sc = _make_sc_kernel(B, F)
    return sc(w_flat, ids_t, vals_t, bias16)


# speedup vs baseline: 3.2348x; 1.8583x over previous
import functools

import jax
import jax.numpy as jnp
from jax import lax
from jax.experimental import pallas as pl
from jax.experimental.pallas import tpu as pltpu
from jax.experimental.pallas import tpu_sc as plsc

NC = 2
NS = 16
NW = NC * NS
L = 16


def _make_sc_kernel(B, F, V):
    rows_pt = B // NW
    assert rows_pt % 128 == 0
    cpf = rows_pt // 128
    nchunk = F * cpf
    groups = rows_pt // L

    mesh = plsc.VectorSubcoreMesh(core_axis_name="c", subcore_axis_name="s")

    slab = (V // NS) & ~63
    tail0 = V - NS * slab

    @functools.partial(
        pl.kernel,
        out_type=jax.ShapeDtypeStruct((B,), jnp.float32),
        mesh=mesh,
        scratch_types=[
            pltpu.VMEM_SHARED((V,), jnp.float32),
            pltpu.VMEM((slab // 8,), jnp.float32),
            pltpu.VMEM((slab // 8,), jnp.float32),
            pltpu.VMEM((F, rows_pt), jnp.int32),
            pltpu.VMEM((F, rows_pt), jnp.float32),
            pltpu.VMEM((F, rows_pt), jnp.float32),
            pltpu.VMEM((L,), jnp.float32),
            pltpu.VMEM((rows_pt,), jnp.float32),
            pltpu.SemaphoreType.DMA,
        ],
    )
    def sc_kernel(w_hbm, ids_hbm, vals_hbm, bias_hbm, out_hbm,
                  spw, bounce, bounce2, idx_v, vals_v, wg_v, bias_v, out_v, sem):
        cid = lax.axis_index("c")
        sid = lax.axis_index("s")
        wid = sid * NC + cid
        b0 = wid * rows_pt

        s0 = sid * slab
        q = slab // 8
        bufs = (bounce, bounce2)
        sem2 = sem
        pltpu.async_copy(w_hbm.at[pl.ds(s0, q)], bounce, sem)
        pltpu.async_copy(w_hbm.at[pl.ds(s0 + q, q)], bounce2, sem2)
        pltpu.sync_copy(ids_hbm.at[:, pl.ds(b0, rows_pt)], idx_v)
        pltpu.sync_copy(vals_hbm.at[:, pl.ds(b0, rows_pt)], vals_v)
        pltpu.sync_copy(bias_hbm, bias_v)
        for r in range(8):
            buf = bufs[r % 2]
            pltpu.make_async_copy(w_hbm.at[pl.ds(s0 + r * q, q)], buf, sem).wait()
            if r + 2 < 8:
                pltpu.async_copy(w_hbm.at[pl.ds(s0 + (r + 2) * q, q)], buf, sem)
            pltpu.sync_copy(buf, spw.at[pl.ds(s0 + r * q, q)])

        if tail0:
            @pl.when(sid == 0)
            def _():
                pltpu.async_copy(w_hbm.at[pl.ds(NS * slab, tail0)],
                                 bounce.at[pl.ds(0, tail0)], sem).wait()
                pltpu.sync_copy(bounce.at[pl.ds(0, tail0)],
                                spw.at[pl.ds(NS * slab, tail0)])

        plsc.subcore_barrier()

        def fire(j, carry):
            f = j // cpf
            c = (j % cpf) * 128
            pltpu.async_copy(
                spw.at[idx_v.at[f, pl.ds(c, 128)]],
                wg_v.at[f, pl.ds(c, 128)], sem)
            return carry

        lax.fori_loop(0, nchunk, fire, 0)

        def drain(j, carry):
            f = j // cpf
            c = (j % cpf) * 128
            pltpu.make_async_copy(
                spw.at[idx_v.at[f, pl.ds(c, 128)]],
                wg_v.at[f, pl.ds(c, 128)], sem).wait()
            return carry

        lax.fori_loop(0, nchunk, drain, 0)

        bias_vec = bias_v[...]

        def group(g, carry):
            o = g * L
            acc = bias_vec
            for f in range(F):
                acc = acc + wg_v[f, pl.ds(o, L)] * vals_v[f, pl.ds(o, L)]
            out_v[pl.ds(o, L)] = acc
            return carry

        lax.fori_loop(0, groups, group, 0)

        pltpu.sync_copy(out_v, out_hbm.at[pl.ds(b0, rows_pt)])

    return sc_kernel


def kernel(ids, vals, weight, bias):
    B, F = ids.shape
    V = weight.shape[0]
    pad = (-V) % 1024
    w_flat = jnp.pad(weight, ((0, pad), (0, 0))).reshape(-1)
    ids_t = ids.T
    vals_t = vals.T
    bias16 = jnp.broadcast_to(bias.astype(jnp.float32), (L,))
    sc = _make_sc_kernel(B, F, V + pad)
    return sc(w_flat, ids_t, vals_t, bias16)
